# Initial kernel scaffold; baseline (speedup 1.0000x reference)
import jax
import jax.numpy as jnp
from jax.experimental import pallas as pl


def kernel(r_idx, r_weight, kv):
    raise NotImplementedError("write your pallas kernel here")


# speedup vs baseline: 2.7881x; 2.7881x over previous
import functools

import jax
import jax.numpy as jnp
from jax import lax
from jax.experimental import pallas as pl
from jax.experimental.pallas import tpu as pltpu
from jax.experimental.pallas import tpu_sc as plsc

_NC = 2
_NS = 16
_NW = _NC * _NS
_L = 16
_CH = 4


def _sc_gather_kernel(Q, D, NCHUNK, gidx_hbm, w_hbm, kv_hbm, out_hbm,
                      idx_v, w_v, buf_v, sem):
    wid = lax.axis_index("s") * _NC + lax.axis_index("c")
    pltpu.sync_copy(gidx_hbm.at[wid], idx_v)
    pltpu.sync_copy(w_hbm.at[wid], w_v)
    base_out = wid * Q

    full = NCHUNK - 1
    tail = Q - full * _CH

    def scale_row(c, j):
        wsp = w_v[c * _CH + j]

        def mul_body(s, _):
            off = s * (4 * _L)
            for u in range(4):
                sl = pl.ds(off + u * _L, _L)
                buf_v[j, sl] = buf_v[j, sl] * wsp
            return 0

        lax.fori_loop(0, D // (4 * _L), mul_body, 0)

    def chunk_body(c, _):
        pltpu.async_copy(kv_hbm.at[idx_v.at[c]], buf_v, sem).wait()
        for j in range(_CH):
            scale_row(c, j)
        pltpu.sync_copy(buf_v, out_hbm.at[pl.ds(base_out + c * _CH, _CH)])
        return 0

    lax.fori_loop(0, full, chunk_body, 0)
    pltpu.async_copy(kv_hbm.at[idx_v.at[full]], buf_v, sem).wait()
    for j in range(tail):
        scale_row(full, j)
    pltpu.sync_copy(buf_v.at[pl.ds(0, tail)],
                    out_hbm.at[pl.ds(base_out + full * _CH, tail)])


def kernel(r_idx, r_weight, kv):
    n, p2, w2, c_kv = kv.shape
    topk = r_idx.shape[-1]
    R = n * p2
    D = w2 * c_kv
    nrows = R * topk
    assert nrows % _NW == 0
    Q = nrows // _NW
    nchunk = -(-Q // _CH)
    qp = 16 * (-(-nchunk * _CH // 16))
    qpad = qp - Q

    kv_flat = kv.reshape(R, D)
    gidx = (jnp.arange(n, dtype=jnp.int32)[:, None, None] * p2
            + r_idx).reshape(_NW, Q)
    w_all = r_weight.reshape(_NW, Q)
    gidx_p = jnp.pad(gidx, ((0, 0), (0, qpad))).reshape(_NW, qp // _CH, _CH)
    w_p = jnp.broadcast_to(
        jnp.pad(w_all, ((0, 0), (0, qpad)))[:, :, None], (_NW, qp, _L))

    mesh = plsc.VectorSubcoreMesh(core_axis_name="c", subcore_axis_name="s")
    body = functools.partial(_sc_gather_kernel, Q, D, nchunk)
    out = pl.kernel(
        body,
        out_type=jax.ShapeDtypeStruct((nrows, D), jnp.float32),
        mesh=mesh,
        compiler_params=pltpu.CompilerParams(use_tc_tiling_on_sc=False),
        scratch_types=[
            pltpu.VMEM((qp // _CH, _CH), jnp.int32),
            pltpu.VMEM((qp, _L), jnp.float32),
            pltpu.VMEM((_CH, D), jnp.float32),
            pltpu.SemaphoreType.DMA,
        ],
    )(gidx_p, w_p, kv_flat)
    return out.reshape(n, p2, topk, w2, c_kv)
